# Initial kernel scaffold; baseline (speedup 1.0000x reference)
import jax
import jax.numpy as jnp
from jax.experimental import pallas as pl


def kernel(x, edge_index, W1, b1, W2, b2):
    raise NotImplementedError("write your pallas kernel here")


# speedup vs baseline: 12.5229x; 12.5229x over previous
import functools

import jax
import jax.numpy as jnp
from jax import lax
from jax.experimental import pallas as pl
from jax.experimental.pallas import tpu as pltpu
from jax.experimental.pallas import tpu_sc as plsc

N = 10000
E = 320000
D = 128

NC = 2
NS = 16
NW = NC * NS
EPW = E // NW
CH = 80
NCH = EPW // CH
NROWS = 10240
RPT = NROWS // NS
ZR = 128
DPT = 640
NPAD = NS * DPT

f32 = jnp.float32

_mesh = plsc.VectorSubcoreMesh(
    core_axis_name="c", subcore_axis_name="s", num_cores=NC, num_subcores=NS
)


def _deg_body(dst_hbm, out_hbm, idx_v, ones_v, zb_v, dacc, sem):
    c = lax.axis_index("c")
    s = lax.axis_index("s")
    wid = s * NC + c

    @pl.loop(0, CH // 16)
    def _(j):
        ones_v[pl.ds(j * 16, 16)] = jnp.ones((16,), f32)

    @pl.loop(0, DPT // 16)
    def _(j):
        zb_v[pl.ds(j * 16, 16)] = jnp.zeros((16,), f32)

    pltpu.sync_copy(zb_v, dacc.at[pl.ds(s * DPT, DPT)])
    plsc.subcore_barrier()

    base = wid * EPW

    @pl.loop(0, NCH)
    def _(i):
        pltpu.sync_copy(dst_hbm.at[pl.ds(base + i * CH, CH)], idx_v)
        pltpu.sync_copy(ones_v, dacc.at[idx_v], add=True)

    plsc.subcore_barrier()
    pltpu.sync_copy(dacc.at[pl.ds(s * DPT, DPT)],
                    out_hbm.at[c, pl.ds(s * DPT, DPT)])


@jax.jit
def _sc_degree(dst):
    return pl.kernel(
        _deg_body,
        out_type=jax.ShapeDtypeStruct((NC, NPAD), f32),
        mesh=_mesh,
        scratch_types=[
            pltpu.VMEM((CH,), jnp.int32),
            pltpu.VMEM((CH,), f32),
            pltpu.VMEM((DPT,), f32),
            pltpu.VMEM_SHARED((NPAD,), f32),
            pltpu.SemaphoreType.DMA,
        ],
    )(dst)


def _segsum_body(y_hbm, src_hbm, dst_hbm, out_hbm,
                 sidx, didx, rows, zb, acc, sem):
    c = lax.axis_index("c")
    s = lax.axis_index("s")
    wid = s * NC + c

    @pl.loop(0, ZR)
    def _(r):
        for j in range(D // 16):
            zb[r, pl.ds(j * 16, 16)] = jnp.zeros((16,), f32)

    @pl.loop(0, RPT // ZR)
    def _(k):
        pltpu.sync_copy(zb, acc.at[pl.ds(s * RPT + k * ZR, ZR)])

    plsc.subcore_barrier()

    base = wid * EPW

    @pl.loop(0, NCH)
    def _(i):
        pltpu.sync_copy(src_hbm.at[pl.ds(base + i * CH, CH)], sidx)
        pltpu.sync_copy(dst_hbm.at[pl.ds(base + i * CH, CH)], didx)
        pltpu.async_copy(y_hbm.at[sidx], rows, sem).wait()
        pltpu.sync_copy(rows, acc.at[didx], add=True)

    plsc.subcore_barrier()
    pltpu.sync_copy(acc.at[pl.ds(s * RPT, RPT)],
                    out_hbm.at[c, pl.ds(s * RPT, RPT)])


@jax.jit
def _sc_segsum(y, src, dst):
    return pl.kernel(
        _segsum_body,
        out_type=jax.ShapeDtypeStruct((NC, NROWS, D), f32),
        mesh=_mesh,
        scratch_types=[
            pltpu.VMEM((CH,), jnp.int32),
            pltpu.VMEM((CH,), jnp.int32),
            pltpu.VMEM((CH, D), f32),
            pltpu.VMEM((ZR, D), f32),
            pltpu.VMEM_SHARED((NROWS, D), f32),
            pltpu.SemaphoreType.DMA,
        ],
    )(y, src, dst)


BR = 400
GRID = N // BR


def _s1_body(x_ref, w_ref, d0_ref, d1_ref, xw_ref, y_ref, dinv_ref):
    xw = jnp.dot(x_ref[...], w_ref[...], preferred_element_type=f32)
    dinv = lax.rsqrt(d0_ref[...] + d1_ref[...] + 1.0)
    xw_ref[...] = xw
    y_ref[...] = xw * dinv
    dinv_ref[...] = dinv


@jax.jit
def _tc_stage1(x, W1, d0, d1):
    return pl.pallas_call(
        _s1_body,
        grid=(GRID,),
        in_specs=[
            pl.BlockSpec((BR, D), lambda i: (i, 0)),
            pl.BlockSpec((D, D), lambda i: (0, 0)),
            pl.BlockSpec((BR, 1), lambda i: (i, 0)),
            pl.BlockSpec((BR, 1), lambda i: (i, 0)),
        ],
        out_specs=[
            pl.BlockSpec((BR, D), lambda i: (i, 0)),
            pl.BlockSpec((BR, D), lambda i: (i, 0)),
            pl.BlockSpec((BR, 1), lambda i: (i, 0)),
        ],
        out_shape=[
            jax.ShapeDtypeStruct((N, D), f32),
            jax.ShapeDtypeStruct((N, D), f32),
            jax.ShapeDtypeStruct((N, 1), f32),
        ],
    )(x, W1, d0, d1)


def _s2_body(s0_ref, s1_ref, xw1_ref, dinv_ref, b1_ref, w2_ref,
             xw2_ref, y2_ref):
    dinv = dinv_ref[...]
    h = (s0_ref[...] + s1_ref[...]) * dinv \
        + xw1_ref[...] * (dinv * dinv) + b1_ref[...]
    h = jnp.maximum(h, 0.0)
    xw2 = jnp.dot(h, w2_ref[...], preferred_element_type=f32)
    xw2_ref[...] = xw2
    y2_ref[...] = xw2 * dinv


@jax.jit
def _tc_stage2(s0, s1, xw1, dinv, b1, W2):
    return pl.pallas_call(
        _s2_body,
        grid=(GRID,),
        in_specs=[
            pl.BlockSpec((BR, D), lambda i: (i, 0)),
            pl.BlockSpec((BR, D), lambda i: (i, 0)),
            pl.BlockSpec((BR, D), lambda i: (i, 0)),
            pl.BlockSpec((BR, 1), lambda i: (i, 0)),
            pl.BlockSpec((1, D), lambda i: (0, 0)),
            pl.BlockSpec((D, D), lambda i: (0, 0)),
        ],
        out_specs=[
            pl.BlockSpec((BR, D), lambda i: (i, 0)),
            pl.BlockSpec((BR, D), lambda i: (i, 0)),
        ],
        out_shape=[
            jax.ShapeDtypeStruct((N, D), f32),
            jax.ShapeDtypeStruct((N, D), f32),
        ],
    )(s0, s1, xw1, dinv, b1, W2)


def _s3_body(s0_ref, s1_ref, xw2_ref, dinv_ref, b2_ref, out_ref):
    dinv = dinv_ref[...]
    out_ref[...] = (s0_ref[...] + s1_ref[...]) * dinv \
        + xw2_ref[...] * (dinv * dinv) + b2_ref[...]


@jax.jit
def _tc_stage3(s0, s1, xw2, dinv, b2):
    return pl.pallas_call(
        _s3_body,
        grid=(GRID,),
        in_specs=[
            pl.BlockSpec((BR, D), lambda i: (i, 0)),
            pl.BlockSpec((BR, D), lambda i: (i, 0)),
            pl.BlockSpec((BR, D), lambda i: (i, 0)),
            pl.BlockSpec((BR, 1), lambda i: (i, 0)),
            pl.BlockSpec((1, D), lambda i: (0, 0)),
        ],
        out_specs=pl.BlockSpec((BR, D), lambda i: (i, 0)),
        out_shape=jax.ShapeDtypeStruct((N, D), f32),
    )(s0, s1, xw2, dinv, b2)


def kernel(x, edge_index, W1, b1, W2, b2):
    src = edge_index[0].astype(jnp.int32)
    dst = edge_index[1].astype(jnp.int32)

    degp = _sc_degree(dst)
    d0 = degp[0, :N].reshape(N, 1)
    d1 = degp[1, :N].reshape(N, 1)

    xw1, y1, dinv = _tc_stage1(x, W1, d0, d1)

    s1p = _sc_segsum(y1, src, dst)
    xw2, y2 = _tc_stage2(s1p[0, :N], s1p[1, :N], xw1, dinv,
                         b1.reshape(1, D), W2)

    s2p = _sc_segsum(y2, src, dst)
    out = _tc_stage3(s2p[0, :N], s2p[1, :N], xw2, dinv, b2.reshape(1, D))
    return out
